# baseline (device time: 38770 ns/iter reference)
import jax
import jax.numpy as jnp
from jax import lax
from jax.experimental import pallas as pl
from jax.experimental.pallas import tpu as pltpu

M = 1024
N = 512
DR, DC = 8, 128


def kernel(x, dest):
    def body(x_ref, d_ref, xall_ref, dall_ref, send_sems, recv_sems):
        my_x = lax.axis_index("x")
        my_y = lax.axis_index("y")
        peer = (1 - my_x, my_y)

        barrier_sem = pltpu.get_barrier_semaphore()
        pl.semaphore_signal(
            barrier_sem, inc=1, device_id=peer,
            device_id_type=pl.DeviceIdType.MESH,
        )
        pl.semaphore_wait(barrier_sem, 1)

        def exchange(slot):
            xall_ref[slot] = x_ref[...]
            dall_ref[slot] = d_ref[...]
            rx = pltpu.make_async_remote_copy(
                src_ref=x_ref,
                dst_ref=xall_ref.at[slot],
                send_sem=send_sems.at[0],
                recv_sem=recv_sems.at[0],
                device_id=peer,
                device_id_type=pl.DeviceIdType.MESH,
            )
            rd = pltpu.make_async_remote_copy(
                src_ref=d_ref,
                dst_ref=dall_ref.at[slot],
                send_sem=send_sems.at[1],
                recv_sem=recv_sems.at[1],
                device_id=peer,
                device_id_type=pl.DeviceIdType.MESH,
            )
            rx.start()
            rd.start()
            rx.wait()
            rd.wait()

        @pl.when(my_x == 0)
        def _():
            exchange(0)

        @pl.when(my_x == 1)
        def _():
            exchange(1)

    x_all, d_all = pl.pallas_call(
        body,
        out_shape=(
            jax.ShapeDtypeStruct((2, M, N), jnp.float32),
            jax.ShapeDtypeStruct((2, DR, DC), jnp.int32),
        ),
        in_specs=[
            pl.BlockSpec(memory_space=pltpu.VMEM),
            pl.BlockSpec(memory_space=pltpu.VMEM),
        ],
        out_specs=(
            pl.BlockSpec(memory_space=pltpu.VMEM),
            pl.BlockSpec(memory_space=pltpu.VMEM),
        ),
        scratch_shapes=[
            pltpu.SemaphoreType.DMA((2,)),
            pltpu.SemaphoreType.DMA((2,)),
        ],
        compiler_params=pltpu.CompilerParams(collective_id=0),
    )(x, dest.reshape(DR, DC))

    x_glob = x_all.reshape(2 * M, N)
    d_glob = d_all.reshape(2 * M)
    order = jnp.argsort(d_glob, stable=True)
    my_x = lax.axis_index("x")
    block = lax.dynamic_slice_in_dim(order, my_x * M, M)
    return jnp.take(x_glob, block, axis=0)


# device time: 29067 ns/iter; 1.3338x vs baseline; 1.3338x over previous
import jax
import jax.numpy as jnp
from jax import lax
from jax.experimental import pallas as pl
from jax.experimental.pallas import tpu as pltpu

M = 1024
N = 512
C = 128
MAX_CHUNKS = M // C


def kernel(x, dest):
    order = jnp.argsort(dest, stable=True)
    xs = jnp.take(x, order, axis=0)
    c0 = jnp.sum(dest == 0).astype(jnp.int32)

    def body(c_ref, xs_ref, stg_ref, send_sems, recv_sems):
        my_x = lax.axis_index("x")
        my_y = lax.axis_index("y")
        peer = (1 - my_x, my_y)

        c = c_ref[0]
        is0 = my_x == 0
        src_start = jnp.where(is0, c, 0)
        src_al = (src_start // 8) * 8
        n_send = jnp.where(is0, M - c, c)
        total8 = ((n_send + (src_start - src_al) + 7) // 8) * 8
        n_cs = (total8 + C - 1) // C
        peer_c0 = M - c
        peer_src_start = jnp.where(is0, 0, peer_c0)
        peer_lead = peer_src_start % 8
        peer_total8 = ((n_send + peer_lead + 7) // 8) * 8
        n_cr = (peer_total8 + C - 1) // C

        barrier_sem = pltpu.get_barrier_semaphore()
        pl.semaphore_signal(
            barrier_sem, inc=1, device_id=peer,
            device_id_type=pl.DeviceIdType.MESH,
        )
        pl.semaphore_wait(barrier_sem, 1)

        for j in range(MAX_CHUNKS):
            @pl.when(j < n_cs)
            def _(j=j):
                off = jnp.minimum(j * C, total8 - C)
                pltpu.make_async_remote_copy(
                    src_ref=xs_ref.at[pl.ds(src_al + off, C)],
                    dst_ref=stg_ref.at[pl.ds(off, C)],
                    send_sem=send_sems.at[j],
                    recv_sem=recv_sems.at[j],
                    device_id=peer,
                    device_id_type=pl.DeviceIdType.MESH,
                ).start()

        for j in range(MAX_CHUNKS):
            @pl.when(j < n_cs)
            def _(j=j):
                pltpu.make_async_remote_copy(
                    src_ref=xs_ref.at[pl.ds(0, C)],
                    dst_ref=stg_ref.at[pl.ds(0, C)],
                    send_sem=send_sems.at[j],
                    recv_sem=recv_sems.at[j],
                    device_id=peer,
                    device_id_type=pl.DeviceIdType.MESH,
                ).wait_send()
        for j in range(MAX_CHUNKS):
            @pl.when(j < n_cr)
            def _(j=j):
                pltpu.make_async_remote_copy(
                    src_ref=xs_ref.at[pl.ds(0, C)],
                    dst_ref=stg_ref.at[pl.ds(0, C)],
                    send_sem=send_sems.at[j],
                    recv_sem=recv_sems.at[j],
                    device_id=peer,
                    device_id_type=pl.DeviceIdType.MESH,
                ).wait_recv()

    staging = pl.pallas_call(
        body,
        out_shape=jax.ShapeDtypeStruct((M, N), jnp.float32),
        in_specs=[
            pl.BlockSpec(memory_space=pltpu.SMEM),
            pl.BlockSpec(memory_space=pltpu.VMEM),
        ],
        out_specs=pl.BlockSpec(memory_space=pltpu.VMEM),
        scratch_shapes=[
            pltpu.SemaphoreType.DMA((MAX_CHUNKS,)),
            pltpu.SemaphoreType.DMA((MAX_CHUNKS,)),
        ],
        compiler_params=pltpu.CompilerParams(collective_id=0),
    )(c0.reshape(1), xs)

    my_x = lax.axis_index("x")
    is0 = my_x == 0
    lead1 = (M - c0) % 8
    shift = jnp.where(is0, c0, -lead1)
    rolled = jnp.roll(staging, shift, axis=0)
    i = jnp.arange(M)
    keep_mask = (i < c0) == is0
    return jnp.where(keep_mask[:, None], xs, rolled)


# device time: 28837 ns/iter; 1.3445x vs baseline; 1.0080x over previous
import jax
import jax.numpy as jnp
from jax import lax
from jax.experimental import pallas as pl
from jax.experimental.pallas import tpu as pltpu

M = 1024
N = 512
C = 128
MAX_CHUNKS = M // C


def kernel(x, dest):
    order = jnp.argsort(dest, stable=True)
    xs = jnp.take(x, order, axis=0)
    c0 = jnp.sum(dest == 0).astype(jnp.int32)

    def body(c_ref, xs_ref, out_ref, stg_ref, send_sems, recv_sems):
        my_x = lax.axis_index("x")
        my_y = lax.axis_index("y")
        peer = (1 - my_x, my_y)

        c = c_ref[0]
        is0 = my_x == 0
        src_start = jnp.where(is0, c, 0)
        src_al = (src_start // 8) * 8
        n_send = jnp.where(is0, M - c, c)
        total8 = ((n_send + (src_start - src_al) + 7) // 8) * 8
        n_cs = (total8 + C - 1) // C
        peer_c0 = M - c
        peer_src_start = jnp.where(is0, 0, peer_c0)
        peer_lead = peer_src_start % 8
        peer_total8 = ((n_send + peer_lead + 7) // 8) * 8
        n_cr = (peer_total8 + C - 1) // C

        barrier_sem = pltpu.get_barrier_semaphore()
        pl.semaphore_signal(
            barrier_sem, inc=1, device_id=peer,
            device_id_type=pl.DeviceIdType.MESH,
        )
        pl.semaphore_wait(barrier_sem, 1)

        for j in range(MAX_CHUNKS):
            @pl.when(j < n_cs)
            def _(j=j):
                off = jnp.minimum(j * C, total8 - C)
                pltpu.make_async_remote_copy(
                    src_ref=xs_ref.at[pl.ds(src_al + off, C)],
                    dst_ref=stg_ref.at[pl.ds(off, C)],
                    send_sem=send_sems.at[j],
                    recv_sem=recv_sems.at[j],
                    device_id=peer,
                    device_id_type=pl.DeviceIdType.MESH,
                ).start()

        for j in range(MAX_CHUNKS):
            @pl.when(j < n_cs)
            def _(j=j):
                pltpu.make_async_remote_copy(
                    src_ref=xs_ref.at[pl.ds(0, C)],
                    dst_ref=stg_ref.at[pl.ds(0, C)],
                    send_sem=send_sems.at[j],
                    recv_sem=recv_sems.at[j],
                    device_id=peer,
                    device_id_type=pl.DeviceIdType.MESH,
                ).wait_send()
        for j in range(MAX_CHUNKS):
            @pl.when(j < n_cr)
            def _(j=j):
                pltpu.make_async_remote_copy(
                    src_ref=xs_ref.at[pl.ds(0, C)],
                    dst_ref=stg_ref.at[pl.ds(0, C)],
                    send_sem=send_sems.at[j],
                    recv_sem=recv_sems.at[j],
                    device_id=peer,
                    device_id_type=pl.DeviceIdType.MESH,
                ).wait_recv()

        lead = (M - c) % 8
        shift = jnp.where(is0, c, (M - lead) % M)
        rolled = pltpu.roll(stg_ref[...], shift, 0)
        idx = lax.broadcasted_iota(jnp.int32, (M, 1), 0)
        keep_mask = (idx < c) == is0
        out_ref[...] = jnp.where(keep_mask, xs_ref[...], rolled)

    out = pl.pallas_call(
        body,
        out_shape=jax.ShapeDtypeStruct((M, N), jnp.float32),
        in_specs=[
            pl.BlockSpec(memory_space=pltpu.SMEM),
            pl.BlockSpec(memory_space=pltpu.VMEM),
        ],
        out_specs=pl.BlockSpec(memory_space=pltpu.VMEM),
        scratch_shapes=[
            pltpu.VMEM((M, N), jnp.float32),
            pltpu.SemaphoreType.DMA((MAX_CHUNKS,)),
            pltpu.SemaphoreType.DMA((MAX_CHUNKS,)),
        ],
        compiler_params=pltpu.CompilerParams(collective_id=0),
    )(c0.reshape(1), xs)
    return out


# device time: 27217 ns/iter; 1.4245x vs baseline; 1.0595x over previous
import jax
import jax.numpy as jnp
from jax import lax
from jax.experimental import pallas as pl
from jax.experimental.pallas import tpu as pltpu

M = 1024
N = 512
C = 128
MAX_CHUNKS = M // C


def kernel(x, dest):
    d0 = dest == 0
    cz = jnp.cumsum(d0.astype(jnp.int32))
    c0 = cz[-1].astype(jnp.int32)
    i = jnp.arange(M, dtype=jnp.int32)
    p = jnp.where(d0, cz - 1, c0 + i - cz)
    xs = jnp.zeros_like(x).at[p].set(
        x, unique_indices=True, mode="promise_in_bounds"
    )

    def body(c_ref, xs_ref, out_ref, stg_ref, send_sems, recv_sems):
        my_x = lax.axis_index("x")
        my_y = lax.axis_index("y")
        peer = (1 - my_x, my_y)

        c = c_ref[0]
        is0 = my_x == 0
        src_start = jnp.where(is0, c, 0)
        src_al = (src_start // 8) * 8
        n_send = jnp.where(is0, M - c, c)
        total8 = ((n_send + (src_start - src_al) + 7) // 8) * 8
        n_cs = (total8 + C - 1) // C
        peer_c0 = M - c
        peer_src_start = jnp.where(is0, 0, peer_c0)
        peer_lead = peer_src_start % 8
        peer_total8 = ((n_send + peer_lead + 7) // 8) * 8
        n_cr = (peer_total8 + C - 1) // C

        barrier_sem = pltpu.get_barrier_semaphore()
        pl.semaphore_signal(
            barrier_sem, inc=1, device_id=peer,
            device_id_type=pl.DeviceIdType.MESH,
        )
        pl.semaphore_wait(barrier_sem, 1)

        for j in range(MAX_CHUNKS):
            @pl.when(j < n_cs)
            def _(j=j):
                off = jnp.minimum(j * C, total8 - C)
                pltpu.make_async_remote_copy(
                    src_ref=xs_ref.at[pl.ds(src_al + off, C)],
                    dst_ref=stg_ref.at[pl.ds(off, C)],
                    send_sem=send_sems.at[j],
                    recv_sem=recv_sems.at[j],
                    device_id=peer,
                    device_id_type=pl.DeviceIdType.MESH,
                ).start()

        for j in range(MAX_CHUNKS):
            @pl.when(j < n_cs)
            def _(j=j):
                pltpu.make_async_remote_copy(
                    src_ref=xs_ref.at[pl.ds(0, C)],
                    dst_ref=stg_ref.at[pl.ds(0, C)],
                    send_sem=send_sems.at[j],
                    recv_sem=recv_sems.at[j],
                    device_id=peer,
                    device_id_type=pl.DeviceIdType.MESH,
                ).wait_send()
        for j in range(MAX_CHUNKS):
            @pl.when(j < n_cr)
            def _(j=j):
                pltpu.make_async_remote_copy(
                    src_ref=xs_ref.at[pl.ds(0, C)],
                    dst_ref=stg_ref.at[pl.ds(0, C)],
                    send_sem=send_sems.at[j],
                    recv_sem=recv_sems.at[j],
                    device_id=peer,
                    device_id_type=pl.DeviceIdType.MESH,
                ).wait_recv()

        lead = (M - c) % 8
        shift = jnp.where(is0, c, (M - lead) % M)
        rolled = pltpu.roll(stg_ref[...], shift, 0)
        idx = lax.broadcasted_iota(jnp.int32, (M, 1), 0)
        keep_mask = (idx < c) == is0
        out_ref[...] = jnp.where(keep_mask, xs_ref[...], rolled)

    out = pl.pallas_call(
        body,
        out_shape=jax.ShapeDtypeStruct((M, N), jnp.float32),
        in_specs=[
            pl.BlockSpec(memory_space=pltpu.SMEM),
            pl.BlockSpec(memory_space=pltpu.VMEM),
        ],
        out_specs=pl.BlockSpec(memory_space=pltpu.VMEM),
        scratch_shapes=[
            pltpu.VMEM((M, N), jnp.float32),
            pltpu.SemaphoreType.DMA((MAX_CHUNKS,)),
            pltpu.SemaphoreType.DMA((MAX_CHUNKS,)),
        ],
        compiler_params=pltpu.CompilerParams(collective_id=0),
    )(c0.reshape(1), xs)
    return out
